# baseline (speedup 1.0000x reference)
import functools

import jax
import jax.numpy as jnp
from jax import lax
from jax.experimental import pallas as pl
from jax.experimental.pallas import tpu as pltpu
from jax.experimental.pallas import tpu_sc as plsc

NC = 2
NS = 16
NW = NC * NS
CHUNK = 128
EMB = 64
SCALE = 8.0


@functools.partial(jax.jit, static_argnames=("n_chunks",))
def _gather_scale(xw, lut, n_chunks):
    tot = NW * n_chunks * CHUNK
    mesh = plsc.VectorSubcoreMesh(core_axis_name="c", subcore_axis_name="s")

    @functools.partial(
        pl.kernel,
        out_type=jax.ShapeDtypeStruct((tot, EMB), jnp.float32),
        mesh=mesh,
        scratch_types=[
            pltpu.VMEM((n_chunks, CHUNK), jnp.int32),
            pltpu.VMEM((CHUNK, EMB), jnp.float32),
            pltpu.SemaphoreType.DMA,
        ],
        compiler_params=pltpu.CompilerParams(use_tc_tiling_on_sc=False),
    )
    def k(x_hbm, lut_hbm, out_hbm, idx_v, rows_v, sem):
        wid = lax.axis_index("s") * NC + lax.axis_index("c")
        base = wid * n_chunks * CHUNK
        pltpu.sync_copy(x_hbm.at[wid], idx_v)

        def body(g, carry):
            pltpu.async_copy(lut_hbm.at[idx_v.at[g]], rows_v, sem).wait()

            def scale_row(i, c):
                for v in range(EMB // 16):
                    sl = pl.ds(v * 16, 16)
                    rows_v[i, sl] = rows_v[i, sl] * SCALE
                return c

            lax.fori_loop(0, CHUNK, scale_row, 0)
            pltpu.sync_copy(rows_v, out_hbm.at[pl.ds(base + g * CHUNK, CHUNK)])
            return carry

        lax.fori_loop(0, n_chunks, body, 0)

    return k(xw, lut)


def kernel(x, lut):
    B, L = x.shape
    tot = B * L
    n_chunks = tot // (NW * CHUNK)
    xw = x.reshape(NW, n_chunks, CHUNK)
    out = _gather_scale(xw, lut, n_chunks)
    return out.reshape(B, L, EMB)
